# Initial kernel scaffold; baseline (speedup 1.0000x reference)
import jax
import jax.numpy as jnp
from jax.experimental import pallas as pl


def kernel(x, edge_index, edge_type, node_to_graph, W_init, W_msg, b_msg, W_upd, b_upd, W_score, W_val, W_headout, W_mean):
    raise NotImplementedError("write your pallas kernel here")


# speedup vs baseline: 2.9585x; 2.9585x over previous
import functools

import jax
import jax.numpy as jnp
from jax import lax
from jax.experimental import pallas as pl
from jax.experimental.pallas import tpu as pltpu
from jax.experimental.pallas import tpu_sc as plsc

N = 10000
E = 160000
ATOM = 128
H = 128
L = 10
T = 3
G = 200
HEADS = 12
HD = 64
OUT = 512

NC = 2
NS = 16
NW = NC * NS
NPW = 320
NP = NW * NPW
EC = 64
EP = E + EC

_f32 = jnp.float32
_i32 = jnp.int32


WN = 64
NWIN = NPW // WN


def _sc_edge_body(with_deg, p6, ia, ib, ed, off, *refs):
    if with_deg:
        s_out, mx_out, deg_out = refs[:3]
        (off_v, ed_v, ia_v, ib_v, ra, rb, s_win, mx_win, deg_acc,
         sem1, sem2) = refs[3:]
    else:
        s_out, mx_out = refs[:2]
        (off_v, ed_v, ia_v, ib_v, ra, rb, s_win, mx_win,
         sem1, sem2) = refs[2:]

    wid = lax.axis_index("s") * NC + lax.axis_index("c")
    pltpu.sync_copy(off, off_v)
    nlo = wid * NPW

    zero16 = jnp.zeros((16,), _f32)

    def _zero_win(r, c):
        for k in range(8):
            s_win[r, pl.ds(k * 16, 16)] = zero16
            mx_win[r, pl.ds(k * 16, 16)] = zero16
        return c

    lax.fori_loop(0, WN, _zero_win, 0)
    if with_deg:
        def _zero_deg(r, c):
            deg_acc[r, :] = zero16
            return c
        lax.fori_loop(0, NPW, _zero_deg, 0)
        one0 = jnp.where(lax.iota(_i32, 16) == 0, 1.0, 0.0).astype(_f32)

    for w in range(NWIN):
        wbase = wid * NWIN + w
        elo = off_v[pl.ds(wbase, 16)][0]
        ehi = off_v[pl.ds(wbase + 1, 16)][0]
        rbase = nlo + w * WN
        base0 = elo - lax.rem(elo, 8)
        nch = lax.div(ehi - base0 + (EC - 1), EC)

        def _chunk(ci, c):
            base = pl.multiple_of(base0 + ci * EC, 8)
            pltpu.sync_copy(ia.at[pl.ds(base, EC)], ia_v)
            pltpu.sync_copy(ib.at[pl.ds(base, EC)], ib_v)
            pltpu.sync_copy(ed.at[pl.ds(base, EC)], ed_v.at[pl.ds(0, EC)])
            c1 = pltpu.async_copy(p6.at[ia_v], ra, sem1)
            c2 = pltpu.async_copy(p6.at[ib_v], rb, sem2)
            c1.wait()
            c2.wait()
            jlo = jnp.maximum(elo - base, 0)
            jhi = jnp.minimum(ehi - base, EC)

            def _edge(j, cc):
                dloc = ed_v[pl.ds(j, 16)][0] - nlo
                r = dloc - w * WN
                for k in range(8):
                    sl = pl.ds(k * 16, 16)
                    m = jnp.maximum(ra[j, sl] + rb[j, sl], 0.0)
                    s_win[r, sl] = s_win[r, sl] + m
                    mx_win[r, sl] = jnp.maximum(mx_win[r, sl], m)
                if with_deg:
                    plsc.addupdate(deg_acc.at[dloc, :], one0)
                return cc

            lax.fori_loop(jlo, jhi, _edge, 0)
            return c

        lax.fori_loop(0, nch, _chunk, 0)
        pltpu.sync_copy(s_win, s_out.at[pl.ds(rbase, WN)])
        pltpu.sync_copy(mx_win, mx_out.at[pl.ds(rbase, WN)])
        lax.fori_loop(0, WN, _zero_win, 0)

    if with_deg:
        pltpu.sync_copy(deg_acc, deg_out.at[pl.ds(nlo, NPW)])


def _make_sc_edge(with_deg):
    out_type = [
        jax.ShapeDtypeStruct((NP, H), _f32),
        jax.ShapeDtypeStruct((NP, H), _f32),
    ]
    scratch = [
        pltpu.VMEM((176,), _i32),
        pltpu.VMEM((EC + 16,), _i32),
        pltpu.VMEM((EC,), _i32),
        pltpu.VMEM((EC,), _i32),
        pltpu.VMEM((EC, H), _f32),
        pltpu.VMEM((EC, H), _f32),
        pltpu.VMEM((WN, H), _f32),
        pltpu.VMEM((WN, H), _f32),
    ]
    if with_deg:
        out_type.append(jax.ShapeDtypeStruct((NP, 16), _f32))
        scratch.append(pltpu.VMEM((NPW, 16), _f32))
    scratch += [pltpu.SemaphoreType.DMA, pltpu.SemaphoreType.DMA]
    mesh = plsc.VectorSubcoreMesh(core_axis_name="c", subcore_axis_name="s",
                                  num_cores=NC, num_subcores=NS)
    return pl.kernel(
        functools.partial(_sc_edge_body, with_deg),
        out_type=out_type,
        mesh=mesh,
        scratch_types=scratch,
    )


@functools.lru_cache(maxsize=None)
def _get_sc_edge(with_deg):
    return _make_sc_edge(with_deg)


BN = 1000


def _init_body(x, wi, wc, bc, h_out, p_out):
    h = jnp.dot(x[...], wi[...], preferred_element_type=_f32)
    h_out[...] = h
    p_out[...] = jnp.dot(h, wc[...], preferred_element_type=_f32) + bc[...]


def _tc_init(x, wi, wc, bc):
    grid = N // BN
    return pl.pallas_call(
        _init_body,
        grid=(grid,),
        in_specs=[
            pl.BlockSpec((BN, ATOM), lambda i: (i, 0)),
            pl.BlockSpec((ATOM, H), lambda i: (0, 0)),
            pl.BlockSpec((H, 6 * H), lambda i: (0, 0)),
            pl.BlockSpec((1, 6 * H), lambda i: (0, 0)),
        ],
        out_specs=[
            pl.BlockSpec((BN, H), lambda i: (i, 0)),
            pl.BlockSpec((BN, 6 * H), lambda i: (i, 0)),
        ],
        out_shape=[
            jax.ShapeDtypeStruct((N, H), _f32),
            jax.ShapeDtypeStruct((N, 6 * H), _f32),
        ],
    )(x, wi, wc, bc)


def _update_body(project, s, mx, h, amp, att, degc, wu, bu, wc, bc, *outs):
    mean = s[...] / degc[...]
    base = jnp.concatenate([mean, mx[...], s[...]], axis=1)
    scaled = jnp.concatenate([base, base * amp[...], base * att[...]], axis=1)
    u = jnp.dot(scaled, wu[...], preferred_element_type=_f32) + bu[...]
    hn = jnp.maximum(u, 0.0) + h[...]
    outs[0][...] = hn
    if project:
        outs[1][...] = jnp.dot(hn, wc[...], preferred_element_type=_f32) + bc[...]


def _tc_update(s, mx, h, amp, att, degc, wu, bu, wc, bc, project):
    grid = N // BN
    out_specs = [pl.BlockSpec((BN, H), lambda i: (i, 0))]
    out_shape = [jax.ShapeDtypeStruct((N, H), _f32)]
    if project:
        out_specs.append(pl.BlockSpec((BN, 6 * H), lambda i: (i, 0)))
        out_shape.append(jax.ShapeDtypeStruct((N, 6 * H), _f32))
    return pl.pallas_call(
        functools.partial(_update_body, project),
        grid=(grid,),
        in_specs=[
            pl.BlockSpec((BN, H), lambda i: (i, 0)),
            pl.BlockSpec((BN, H), lambda i: (i, 0)),
            pl.BlockSpec((BN, H), lambda i: (i, 0)),
            pl.BlockSpec((BN, 1), lambda i: (i, 0)),
            pl.BlockSpec((BN, 1), lambda i: (i, 0)),
            pl.BlockSpec((BN, 1), lambda i: (i, 0)),
            pl.BlockSpec((9 * H, H), lambda i: (0, 0)),
            pl.BlockSpec((1, H), lambda i: (0, 0)),
            pl.BlockSpec((H, 6 * H), lambda i: (0, 0)),
            pl.BlockSpec((1, 6 * H), lambda i: (0, 0)),
        ],
        out_specs=out_specs,
        out_shape=out_shape,
    )(s, mx, h, amp, att, degc, wu, bu, wc, bc)


def _scaler_body(deg, amp, att, degc):
    d = deg[...]
    ld = jnp.log1p(d)
    delta = jnp.sum(ld) / N
    safe = jnp.where(ld > 0, ld, 1.0)
    amp[...] = ld / delta
    att[...] = delta / safe
    degc[...] = jnp.maximum(d, 1.0)


def _tc_scalers(deg):
    return pl.pallas_call(
        _scaler_body,
        out_shape=[
            jax.ShapeDtypeStruct((N, 1), _f32),
            jax.ShapeDtypeStruct((N, 1), _f32),
            jax.ShapeDtypeStruct((N, 1), _f32),
        ],
    )(deg)


RBN = 1000
DALL = (L + 1) * H


def _readout_body(hb, ntg, ws, wv, wh, wm, out, wacc, hacc, cacc):
    k = pl.program_id(0)
    nblk = pl.num_programs(0)

    @pl.when(k == 0)
    def _():
        wacc[...] = jnp.zeros_like(wacc)
        hacc[...] = jnp.zeros_like(hacc)
        cacc[...] = jnp.zeros_like(cacc)

    hblk = hb[...]
    scores = jax.nn.sigmoid(jnp.dot(hblk, ws[...], preferred_element_type=_f32))
    vals = jnp.dot(hblk, wv[...], preferred_element_type=_f32)
    rep = (lax.broadcasted_iota(_i32, (HEADS, HEADS * HD), 1) // HD ==
           lax.broadcasted_iota(_i32, (HEADS, HEADS * HD), 0)).astype(_f32)
    sv = jnp.dot(scores, rep, preferred_element_type=_f32) * vals
    onehot = (ntg[...] ==
              lax.broadcasted_iota(_i32, (RBN, G), 1)).astype(_f32)
    dn = (((0,), (0,)), ((), ()))
    wacc[...] += lax.dot_general(onehot, sv, dn, preferred_element_type=_f32)
    hacc[...] += lax.dot_general(onehot, hblk, dn, preferred_element_type=_f32)
    cacc[...] += lax.dot_general(onehot, jnp.ones((RBN, 1), _f32), dn,
                                 preferred_element_type=_f32)

    @pl.when(k == nblk - 1)
    def _():
        mean_state = hacc[...] / jnp.maximum(cacc[...], 1.0)
        out[...] = (jnp.dot(wacc[...], wh[...], preferred_element_type=_f32) +
                    jnp.dot(mean_state, wm[...], preferred_element_type=_f32))


def _tc_readout(h_all, ntg, ws, wv, wh, wm):
    grid = N // RBN
    return pl.pallas_call(
        _readout_body,
        grid=(grid,),
        in_specs=[
            pl.BlockSpec((RBN, DALL), lambda i: (i, 0)),
            pl.BlockSpec((RBN, 1), lambda i: (i, 0)),
            pl.BlockSpec((DALL, HEADS), lambda i: (0, 0)),
            pl.BlockSpec((DALL, HEADS * HD), lambda i: (0, 0)),
            pl.BlockSpec((HEADS * HD, OUT), lambda i: (0, 0)),
            pl.BlockSpec((DALL, OUT), lambda i: (0, 0)),
        ],
        out_specs=pl.BlockSpec((G, OUT), lambda i: (0, 0)),
        out_shape=jax.ShapeDtypeStruct((G, OUT), _f32),
        scratch_shapes=[
            pltpu.VMEM((G, HEADS * HD), _f32),
            pltpu.VMEM((G, DALL), _f32),
            pltpu.VMEM((G, 1), _f32),
        ],
    )(h_all, ntg, ws, wv, wh, wm)


def kernel(x, edge_index, edge_type, node_to_graph, W_init, W_msg, b_msg,
           W_upd, b_upd, W_score, W_val, W_headout, W_mean):
    src = edge_index[0]
    dst = edge_index[1]
    order = jnp.argsort(dst)
    ds = dst[order].astype(_i32)
    ss = src[order].astype(_i32)
    ts = edge_type[order].astype(_i32)
    idxa = ss * 6 + ts * 2
    idxb = ds * 6 + ts * 2 + 1
    pad = jnp.zeros((EP - E,), _i32)
    idxa = jnp.concatenate([idxa, pad])
    idxb = jnp.concatenate([idxb, pad])
    ds_p = jnp.concatenate([ds, pad])
    nwoff = NW * (NPW // WN) + 1
    off = jnp.searchsorted(ds, jnp.arange(nwoff, dtype=_i32) * WN)
    off = jnp.concatenate(
        [off.astype(_i32), jnp.zeros((176 - nwoff,), _i32)])

    wm5 = W_msg.reshape(L, T, 2, H, H)
    wcat = wm5.transpose(0, 3, 1, 2, 4).reshape(L, H, 6 * H)
    bcat = jnp.stack([jnp.zeros_like(b_msg), b_msg], axis=2)
    bcat = bcat.reshape(L, 1, 6 * H)
    bu = b_upd.reshape(L, 1, H)

    h, P = _tc_init(x, W_init, wcat[0], bcat[0])
    states = [h]
    amp = att = degc = None
    for l in range(L):
        p6 = P.reshape(6 * N, H)
        if l == 0:
            s_p, mx_p, deg_p = _get_sc_edge(True)(p6, idxa, idxb, ds_p, off)
            amp, att, degc = _tc_scalers(deg_p[:N, :1])
        else:
            s_p, mx_p = _get_sc_edge(False)(p6, idxa, idxb, ds_p, off)
        if l < L - 1:
            h, P = _tc_update(s_p[:N], mx_p[:N], h, amp, att, degc,
                              W_upd[l], bu[l], wcat[l + 1], bcat[l + 1], True)
        else:
            (h,) = _tc_update(s_p[:N], mx_p[:N], h, amp, att, degc,
                              W_upd[l], bu[l], wcat[l], bcat[l], False)
        states.append(h)

    h_all = jnp.concatenate(states, axis=-1)
    ntg = node_to_graph.astype(_i32).reshape(N, 1)
    return _tc_readout(h_all, ntg, W_score, W_val, W_headout, W_mean)
